# baseline (speedup 1.0000x reference)
import functools

import jax
import jax.numpy as jnp
from jax import lax
from jax.experimental import pallas as pl
from jax.experimental.pallas import tpu as pltpu
from jax.experimental.pallas import tpu_sc as plsc

_T = 32768
_E = 64
_NC = 2
_NS = 16
_L = 16
_NW = _NC * _NS
_TPW = _T // _NW
_NG = _TPW // _L

_mesh = plsc.VectorSubcoreMesh(core_axis_name="c", subcore_axis_name="s")


@functools.partial(
    pl.kernel,
    out_type=(
        jax.ShapeDtypeStruct((_T,), jnp.int32),
        jax.ShapeDtypeStruct((_T,), jnp.int32),
    ),
    mesh=_mesh,
    compiler_params=pltpu.CompilerParams(
        needs_layout_passes=False,
        use_tc_tiling_on_sc=True,
        skip_device_barrier=True,
    ),
    scratch_types=[
        pltpu.VMEM((_E, _TPW), jnp.float32),
        pltpu.VMEM((_TPW,), jnp.int32),
        pltpu.VMEM((_TPW,), jnp.int32),
    ],
)
def _route_mask_sc(routet_hbm, nr_hbm, r_hbm, buf, nr_buf, r_buf):
    wid = lax.axis_index("s") * _NC + lax.axis_index("c")
    tbase = wid * _TPW
    pltpu.sync_copy(routet_hbm.at[:, pl.ds(tbase, _TPW)], buf)

    @plsc.parallel_loop(0, _NG, unroll=2)
    def _grp(g):
        col = g * _L
        c0 = buf[0, pl.ds(col, _L)]
        acc = [c0, None, None, None]
        for e in range(1, _E):
            v = buf[e, pl.ds(col, _L)]
            k = e % 4
            acc[k] = v if acc[k] is None else jnp.maximum(acc[k], v)
        m = jnp.maximum(jnp.maximum(acc[0], acc[1]),
                        jnp.maximum(acc[2], acc[3]))
        second = jnp.where(m > c0, 1, 0).astype(jnp.int32)
        r_buf[pl.ds(col, _L)] = second
        nr_buf[pl.ds(col, _L)] = 1 - second

    pltpu.sync_copy(nr_buf, nr_hbm.at[pl.ds(tbase, _TPW)])
    pltpu.sync_copy(r_buf, r_hbm.at[pl.ds(tbase, _TPW)])


def kernel(route, skip_dim):
    nr, r = _route_mask_sc(route.T)
    cond = skip_dim == 1
    first = jnp.where(cond, nr, r).astype(jnp.bool_)
    second = jnp.where(cond, r, nr).astype(jnp.bool_)
    return (first, second)


# speedup vs baseline: 1.0588x; 1.0588x over previous
import functools

import jax
import jax.numpy as jnp
from jax import lax
from jax.experimental import pallas as pl
from jax.experimental.pallas import tpu as pltpu
from jax.experimental.pallas import tpu_sc as plsc

_T = 32768
_E = 64
_NC = 2
_NS = 16
_L = 16
_NW = _NC * _NS
_TPW = _T // _NW
_NG = _TPW // _L

_mesh = plsc.VectorSubcoreMesh(core_axis_name="c", subcore_axis_name="s")


@functools.partial(
    pl.kernel,
    out_type=(
        jax.ShapeDtypeStruct((_T,), jnp.int32),
        jax.ShapeDtypeStruct((_T,), jnp.int32),
    ),
    mesh=_mesh,
    compiler_params=pltpu.CompilerParams(
        needs_layout_passes=False,
        use_tc_tiling_on_sc=True,
    ),
    scratch_types=[
        pltpu.VMEM((_E, _TPW), jnp.float32),
        pltpu.VMEM((_TPW,), jnp.int32),
        pltpu.VMEM((_TPW,), jnp.int32),
    ],
)
def _route_mask_sc(routet_hbm, nr_hbm, r_hbm, buf, nr_buf, r_buf):
    wid = lax.axis_index("s") * _NC + lax.axis_index("c")
    tbase = wid * _TPW
    pltpu.sync_copy(routet_hbm.at[:, pl.ds(tbase, _TPW)], buf)

    @plsc.parallel_loop(0, _NG)
    def _grp(g):
        col = g * _L
        c0 = buf[0, pl.ds(col, _L)]
        acc = [c0, None, None, None]
        for e in range(1, _E):
            v = buf[e, pl.ds(col, _L)]
            k = e % 4
            acc[k] = v if acc[k] is None else jnp.maximum(acc[k], v)
        m = jnp.maximum(jnp.maximum(acc[0], acc[1]),
                        jnp.maximum(acc[2], acc[3]))
        second = jnp.where(m > c0, 1, 0).astype(jnp.int32)
        r_buf[pl.ds(col, _L)] = second
        nr_buf[pl.ds(col, _L)] = 1 - second

    pltpu.sync_copy(nr_buf, nr_hbm.at[pl.ds(tbase, _TPW)])
    pltpu.sync_copy(r_buf, r_hbm.at[pl.ds(tbase, _TPW)])


def kernel(route, skip_dim):
    nr, r = _route_mask_sc(route.T)
    cond = skip_dim == 1
    first = jnp.where(cond, nr, r).astype(jnp.bool_)
    second = jnp.where(cond, r, nr).astype(jnp.bool_)
    return (first, second)


# speedup vs baseline: 1.0700x; 1.0106x over previous
import functools

import jax
import jax.numpy as jnp
from jax import lax
from jax.experimental import pallas as pl
from jax.experimental.pallas import tpu as pltpu
from jax.experimental.pallas import tpu_sc as plsc

_T = 32768
_E = 64
_NC = 2
_NS = 16
_L = 16
_NW = _NC * _NS
_TPW = _T // _NW
_NG = _TPW // _L

_mesh = plsc.VectorSubcoreMesh(core_axis_name="c", subcore_axis_name="s")


@functools.partial(
    pl.kernel,
    out_type=(
        jax.ShapeDtypeStruct((_T,), jnp.int32),
        jax.ShapeDtypeStruct((_T,), jnp.int32),
    ),
    mesh=_mesh,
    compiler_params=pltpu.CompilerParams(
        needs_layout_passes=False,
        use_tc_tiling_on_sc=True,
    ),
    scratch_types=[
        pltpu.VMEM((_E, _TPW), jnp.float32),
        pltpu.VMEM((_TPW,), jnp.int32),
        pltpu.VMEM((_TPW,), jnp.int32),
    ],
)
def _route_mask_sc(routet_hbm, nr_hbm, r_hbm, buf, nr_buf, r_buf):
    wid = lax.axis_index("s") * _NC + lax.axis_index("c")
    tbase = wid * _TPW
    pltpu.sync_copy(routet_hbm.at[:, pl.ds(tbase, _TPW)], buf)

    @plsc.parallel_loop(0, _NG)
    def _grp(g):
        col = g * _L
        c0 = buf[0, pl.ds(col, _L)]

        def _echunk(eb, acc):
            eb8 = eb * 8
            for k in range(8):
                acc[k] = jnp.maximum(acc[k], buf[eb8 + k, pl.ds(col, _L)])
            return acc

        acc = lax.fori_loop(1, _E // 8,
                            _echunk, [buf[k, pl.ds(col, _L)]
                                      for k in range(8)])
        m = acc[0]
        for k in range(1, 8):
            m = jnp.maximum(m, acc[k])
        second = jnp.where(m > c0, 1, 0).astype(jnp.int32)
        r_buf[pl.ds(col, _L)] = second
        nr_buf[pl.ds(col, _L)] = 1 - second

    pltpu.sync_copy(nr_buf, nr_hbm.at[pl.ds(tbase, _TPW)])
    pltpu.sync_copy(r_buf, r_hbm.at[pl.ds(tbase, _TPW)])


def kernel(route, skip_dim):
    nr, r = _route_mask_sc(route.T)
    cond = skip_dim == 1
    first = jnp.where(cond, nr, r).astype(jnp.bool_)
    second = jnp.where(cond, r, nr).astype(jnp.bool_)
    return (first, second)
